# baseline (speedup 1.0000x reference)
import functools

import jax
import jax.numpy as jnp
from jax import lax
from jax.experimental import pallas as pl
from jax.experimental.pallas import tpu as pltpu
from jax.experimental.pallas import tpu_sc as plsc

N = 10000
NPAD = 10240
E = 160000
P = 100000
IN_CH = 256
HID = 512
F = 128
NCHUNK = HID // F
NC, NS = 2, 16
NW = NC * NS

EB = 128
E_PER_TILE = 10240
E_NB = E_PER_TILE // EB
E_NH = E_NB // 2
E_PER_W = 5120
D_NB = E_PER_W // EB
NSTRIPE = NPAD // NS

PB = 32
P_PER_W = 3200
P_NB = P_PER_W // PB
P_REAL = P // NW

_MESH = plsc.VectorSubcoreMesh(core_axis_name="c", subcore_axis_name="s",
                               num_cores=NC, num_subcores=NS)


def _deg_body(dst_hbm, ones_hbm, zeros_hbm, out_hbm, idx_v, ones_v, acc, sem):
    cid = lax.axis_index("c")
    sid = lax.axis_index("s")
    wid = sid * NC + cid
    row0 = sid * NSTRIPE
    pltpu.sync_copy(zeros_hbm.at[pl.ds(row0, NSTRIPE)],
                    acc.at[pl.ds(row0, NSTRIPE)])
    pltpu.sync_copy(ones_hbm, ones_v)
    pltpu.sync_copy(dst_hbm.at[wid], idx_v)
    plsc.subcore_barrier()

    def body(b, carry):
        pltpu.sync_copy(ones_v, acc.at[idx_v.at[b]], add=True)
        return carry
    lax.fori_loop(0, D_NB, body, 0)
    plsc.subcore_barrier()
    pltpu.sync_copy(acc.at[pl.ds(row0, NSTRIPE)],
                    out_hbm.at[cid, pl.ds(row0, NSTRIPE)])


_kdeg = pl.kernel(
    _deg_body,
    out_type=jax.ShapeDtypeStruct((NC, NPAD, F), jnp.float32),
    mesh=_MESH,
    scratch_types=[
        pltpu.VMEM((D_NB, EB), jnp.int32),
        pltpu.VMEM((EB, F), jnp.float32),
        pltpu.VMEM_SHARED((NPAD, F), jnp.float32),
        pltpu.SemaphoreType.DMA,
    ],
)


def _edge_body(srcg_hbm, dste_hbm, hp_hbm, zeros_hbm, out_hbm,
               isrc_v, idst_v, gbuf_a, gbuf_b, acc, sem_a, sem_b,
               sem_sa, sem_sb):
    cid = lax.axis_index("c")
    sid = lax.axis_index("s")
    row0 = sid * NSTRIPE
    for j in range(NCHUNK // NC):
        ch = cid * (NCHUNK // NC) + j
        pltpu.sync_copy(zeros_hbm.at[pl.ds(row0, NSTRIPE)],
                        acc.at[pl.ds(row0, NSTRIPE)])
        plsc.subcore_barrier()
        for h in range(2):
            pltpu.sync_copy(srcg_hbm.at[ch, sid, pl.ds(h * E_NH, E_NH)],
                            isrc_v)
            pltpu.sync_copy(dste_hbm.at[sid, pl.ds(h * E_NH, E_NH)], idst_v)
            pltpu.async_copy(hp_hbm.at[isrc_v.at[0]], gbuf_a, sem_a)
            pltpu.async_copy(hp_hbm.at[isrc_v.at[1]], gbuf_b, sem_b)

            def body(g, carry):
                b = g * 2
                pltpu.make_async_copy(hp_hbm.at[isrc_v.at[b]],
                                      gbuf_a, sem_a).wait()
                pltpu.async_copy(gbuf_a, acc.at[idst_v.at[b]], sem_sa,
                                 add=True)
                pltpu.make_async_copy(hp_hbm.at[isrc_v.at[b + 1]],
                                      gbuf_b, sem_b).wait()
                pltpu.async_copy(gbuf_b, acc.at[idst_v.at[b + 1]], sem_sb,
                                 add=True)
                pltpu.make_async_copy(gbuf_a, acc.at[idst_v.at[b]],
                                      sem_sa).wait()
                pltpu.async_copy(hp_hbm.at[isrc_v.at[lax.rem(b + 2, E_NH)]],
                                 gbuf_a, sem_a)
                pltpu.make_async_copy(gbuf_b, acc.at[idst_v.at[b + 1]],
                                      sem_sb).wait()
                pltpu.async_copy(hp_hbm.at[isrc_v.at[lax.rem(b + 3, E_NH)]],
                                 gbuf_b, sem_b)
                return carry
            lax.fori_loop(0, E_NH // 2, body, 0)
            pltpu.make_async_copy(hp_hbm.at[isrc_v.at[0]],
                                  gbuf_a, sem_a).wait()
            pltpu.make_async_copy(hp_hbm.at[isrc_v.at[1]],
                                  gbuf_b, sem_b).wait()
        plsc.subcore_barrier()
        pltpu.sync_copy(acc.at[pl.ds(row0, NSTRIPE)],
                        out_hbm.at[ch, pl.ds(row0, NSTRIPE)])


_kedge = pl.kernel(
    _edge_body,
    out_type=jax.ShapeDtypeStruct((NCHUNK, NPAD, F), jnp.float32),
    mesh=_MESH,
    scratch_types=[
        pltpu.VMEM((E_NH, EB), jnp.int32),
        pltpu.VMEM((E_NH, EB), jnp.int32),
        pltpu.VMEM((EB, F), jnp.float32),
        pltpu.VMEM((EB, F), jnp.float32),
        pltpu.VMEM_SHARED((NPAD, F), jnp.float32),
        pltpu.SemaphoreType.DMA,
        pltpu.SemaphoreType.DMA,
        pltpu.SemaphoreType.DMA,
        pltpu.SemaphoreType.DMA,
    ],
)


def _dec_compute(ga, gb, wv, res_v):
    for g in range(PB // 16):
        def kstep(k, accs):
            w = wv[pl.ds(k * 16, 16)]
            out = []
            for j in range(16):
                row = g * 16 + j
                av = ga[row, pl.ds(k * 16, 16)]
                bb = gb[row, pl.ds(k * 16, 16)]
                out.append(accs[j] + jnp.maximum(av + bb, 0.0) * w)
            return tuple(out)
        accs = lax.fori_loop(
            0, HID // 16, kstep,
            tuple(jnp.zeros((16,), jnp.float32) for _ in range(16)))
        for j in range(16):
            res_v[g * 16 + j] = accs[j]


def _dec_body(p0_hbm, p1_hbm, a_hbm, b_hbm, wc2_hbm, out_hbm,
              i0_v, i1_v, ga0, gb0, ga1, gb1, wv, res_v,
              sa0, sb0, sa1, sb1):
    cid = lax.axis_index("c")
    sid = lax.axis_index("s")
    wid = sid * NC + cid
    pltpu.sync_copy(p0_hbm.at[wid], i0_v)
    pltpu.sync_copy(p1_hbm.at[wid], i1_v)
    pltpu.sync_copy(wc2_hbm, wv)

    pltpu.async_copy(a_hbm.at[i0_v.at[0]], ga0, sa0)
    pltpu.async_copy(b_hbm.at[i1_v.at[0]], gb0, sb0)
    pltpu.async_copy(a_hbm.at[i0_v.at[1]], ga1, sa1)
    pltpu.async_copy(b_hbm.at[i1_v.at[1]], gb1, sb1)

    def batch(g, carry):
        b = g * 2
        pltpu.make_async_copy(a_hbm.at[i0_v.at[b]], ga0, sa0).wait()
        pltpu.make_async_copy(b_hbm.at[i1_v.at[b]], gb0, sb0).wait()
        _dec_compute(ga0, gb0, wv, res_v)
        pltpu.sync_copy(res_v, out_hbm.at[wid, pl.ds(b * PB, PB)])
        pltpu.async_copy(a_hbm.at[i0_v.at[lax.rem(b + 2, P_NB)]], ga0, sa0)
        pltpu.async_copy(b_hbm.at[i1_v.at[lax.rem(b + 2, P_NB)]], gb0, sb0)
        pltpu.make_async_copy(a_hbm.at[i0_v.at[b + 1]], ga1, sa1).wait()
        pltpu.make_async_copy(b_hbm.at[i1_v.at[b + 1]], gb1, sb1).wait()
        _dec_compute(ga1, gb1, wv, res_v)
        pltpu.sync_copy(res_v, out_hbm.at[wid, pl.ds((b + 1) * PB, PB)])
        pltpu.async_copy(a_hbm.at[i0_v.at[lax.rem(b + 3, P_NB)]], ga1, sa1)
        pltpu.async_copy(b_hbm.at[i1_v.at[lax.rem(b + 3, P_NB)]], gb1, sb1)
        return carry
    lax.fori_loop(0, P_NB // 2, batch, 0)
    pltpu.make_async_copy(a_hbm.at[i0_v.at[0]], ga0, sa0).wait()
    pltpu.make_async_copy(b_hbm.at[i1_v.at[0]], gb0, sb0).wait()
    pltpu.make_async_copy(a_hbm.at[i0_v.at[1]], ga1, sa1).wait()
    pltpu.make_async_copy(b_hbm.at[i1_v.at[1]], gb1, sb1).wait()


_kdec = pl.kernel(
    _dec_body,
    out_type=jax.ShapeDtypeStruct((NW, P_PER_W, 16), jnp.float32),
    mesh=_MESH,
    scratch_types=[
        pltpu.VMEM((P_NB, PB), jnp.int32),
        pltpu.VMEM((P_NB, PB), jnp.int32),
        pltpu.VMEM((PB, HID), jnp.float32),
        pltpu.VMEM((PB, HID), jnp.float32),
        pltpu.VMEM((PB, HID), jnp.float32),
        pltpu.VMEM((PB, HID), jnp.float32),
        pltpu.VMEM((HID,), jnp.float32),
        pltpu.VMEM((PB, 16), jnp.float32),
        pltpu.SemaphoreType.DMA,
        pltpu.SemaphoreType.DMA,
        pltpu.SemaphoreType.DMA,
        pltpu.SemaphoreType.DMA,
    ],
)


def _red_body(p_ref, bc2_ref, o_ref):
    o_ref[...] = jnp.sum(p_ref[...], axis=-1) + bc2_ref[0, 0]


_kred = pl.pallas_call(
    _red_body,
    grid=(NW // 8,),
    in_specs=[pl.BlockSpec((8, P_PER_W, 16), lambda i: (i, 0, 0)),
              pl.BlockSpec((1, 1), lambda i: (0, 0))],
    out_specs=pl.BlockSpec((8, P_PER_W), lambda i: (i, 0)),
    out_shape=jax.ShapeDtypeStruct((NW, P_PER_W), jnp.float32),
)


MBLK = 1000


def _dinv_block(deg3_ref, i):
    degb = deg3_ref[:, pl.ds(i * MBLK, MBLK), :]
    deg = jnp.sum(degb, axis=(0, 2)) * (1.0 / F) + 1.0
    return lax.rsqrt(jnp.maximum(deg, 1.0))


def _mm1_body(x_ref, w_ref, deg3_ref, o_ref):
    i = pl.program_id(0)
    dinv = _dinv_block(deg3_ref, i)
    h = jnp.dot(x_ref[...], w_ref[...], preferred_element_type=jnp.float32)
    o_ref[...] = h * dinv[:, None]


def _mm2_body(s_ref, hp_ref, w_ref, b_ref, deg3_ref, o_ref):
    i = pl.program_id(0)
    dinv = _dinv_block(deg3_ref, i)
    z = jax.nn.relu((s_ref[...] + hp_ref[...]) * dinv[:, None] + b_ref[...])
    h = jnp.dot(z, w_ref[...], preferred_element_type=jnp.float32)
    o_ref[...] = h * dinv[:, None]


def _mm3_body(s_ref, hp_ref, wc1_ref, b_ref, bc1_ref, deg3_ref,
              a_ref, b_out_ref):
    i = pl.program_id(0)
    dinv = _dinv_block(deg3_ref, i)
    z = (s_ref[...] + hp_ref[...]) * dinv[:, None] + b_ref[...]
    a_ref[...] = jnp.dot(z, wc1_ref[pl.ds(0, HID), :],
                         preferred_element_type=jnp.float32) + bc1_ref[...]
    b_out_ref[...] = jnp.dot(z, wc1_ref[pl.ds(HID, HID), :],
                             preferred_element_type=jnp.float32)


def _full_spec(shape):
    nd = len(shape)
    return pl.BlockSpec(shape, lambda i, _nd=nd: (0,) * _nd)


def _row_spec(cols):
    return pl.BlockSpec((MBLK, cols), lambda i: (i, 0))


_GRID = N // MBLK

_kmm1 = pl.pallas_call(
    _mm1_body,
    grid=(_GRID,),
    in_specs=[_row_spec(IN_CH), _full_spec((IN_CH, HID)),
              _full_spec((NC, NPAD, F))],
    out_specs=_row_spec(HID),
    out_shape=jax.ShapeDtypeStruct((N, HID), jnp.float32),
)

_kmm2 = pl.pallas_call(
    _mm2_body,
    grid=(_GRID,),
    in_specs=[_row_spec(HID), _row_spec(HID), _full_spec((HID, HID)),
              _full_spec((1, HID)), _full_spec((NC, NPAD, F))],
    out_specs=_row_spec(HID),
    out_shape=jax.ShapeDtypeStruct((N, HID), jnp.float32),
)

_kmm3 = pl.pallas_call(
    _mm3_body,
    grid=(_GRID,),
    in_specs=[_row_spec(HID), _row_spec(HID), _full_spec((2 * HID, HID)),
              _full_spec((1, HID)), _full_spec((1, HID)),
              _full_spec((NC, NPAD, F))],
    out_specs=[_row_spec(HID), _row_spec(HID)],
    out_shape=[jax.ShapeDtypeStruct((N, HID), jnp.float32),
               jax.ShapeDtypeStruct((N, HID), jnp.float32)],
)


def _prep_edges(edge_index):
    src = edge_index[0].astype(jnp.int32)
    dst = edge_index[1].astype(jnp.int32)
    epad = NS * E_PER_TILE
    src_p = jnp.pad(src, (0, epad - E), constant_values=0)
    dst_p = jnp.pad(dst, (0, epad - E), constant_values=N + 16)
    dst_deg = dst_p.reshape(NW, D_NB, EB)
    dst_edge = dst_p.reshape(NS, E_NB, EB)
    srcg = (src_p[None, :] * NCHUNK
            + jnp.arange(NCHUNK, dtype=jnp.int32)[:, None])
    srcg = srcg.reshape(NCHUNK, NS, E_NB, EB)
    return dst_deg, dst_edge, srcg


def _unchunk(s_c):
    return s_c[:, :N, :].transpose(1, 0, 2).reshape(N, HID)


def kernel(x, edge_index, edge_pairs, W1, b1, W2, b2, Wc1, bc1, Wc2, bc2):
    dst_deg, dst_edge, srcg = _prep_edges(edge_index)
    p0 = edge_pairs[0].astype(jnp.int32).reshape(NW, P_REAL)
    p1 = edge_pairs[1].astype(jnp.int32).reshape(NW, P_REAL)
    p0 = jnp.pad(p0, ((0, 0), (0, P_PER_W - P_REAL))).reshape(NW, P_NB, PB)
    p1 = jnp.pad(p1, ((0, 0), (0, P_PER_W - P_REAL))).reshape(NW, P_NB, PB)

    zeros_big = jnp.zeros((NPAD, F), jnp.float32)
    ones128 = jnp.ones((EB, F), jnp.float32)
    b1r = b1.reshape(1, HID)
    b2r = b2.reshape(1, HID)
    bc1r = bc1.reshape(1, HID)
    wc2 = Wc2.reshape(HID)
    bc2r = bc2.reshape(1, 1)

    deg3 = _kdeg(dst_deg, ones128, zeros_big)
    h1p = _kmm1(x, W1, deg3)
    s1 = _unchunk(_kedge(srcg, dst_edge, h1p.reshape(N * NCHUNK, F), zeros_big))
    h2p = _kmm2(s1, h1p, W2, b1r, deg3)
    s2 = _unchunk(_kedge(srcg, dst_edge, h2p.reshape(N * NCHUNK, F), zeros_big))
    A, B = _kmm3(s2, h2p, Wc1, b2r, bc1r, deg3)
    outp = _kred(_kdec(p0, p1, A, B, wc2), bc2r)
    return outp[:, :P_REAL].reshape(-1)


# speedup vs baseline: 1.0600x; 1.0600x over previous
import functools

import jax
import jax.numpy as jnp
from jax import lax
from jax.experimental import pallas as pl
from jax.experimental.pallas import tpu as pltpu
from jax.experimental.pallas import tpu_sc as plsc

N = 10000
NPAD = 10240
E = 160000
P = 100000
IN_CH = 256
HID = 512
F = 128
NCHUNK = HID // F
NC, NS = 2, 16
NW = NC * NS

EB = 128
E_PER_TILE = 10240
E_NB = E_PER_TILE // EB
E_NH = E_NB // 2
E_PER_W = 5120
D_NB = E_PER_W // EB
NSTRIPE = NPAD // NS

PB = 32
P_PER_W = 3200
P_NB = P_PER_W // PB
P_REAL = P // NW

_MESH = plsc.VectorSubcoreMesh(core_axis_name="c", subcore_axis_name="s",
                               num_cores=NC, num_subcores=NS)


def _deg_body(dst_hbm, ones_hbm, zeros_hbm, out_hbm, idx_v, ones_v, acc, sem):
    cid = lax.axis_index("c")
    sid = lax.axis_index("s")
    wid = sid * NC + cid
    row0 = sid * NSTRIPE
    pltpu.sync_copy(zeros_hbm.at[pl.ds(row0, NSTRIPE)],
                    acc.at[pl.ds(row0, NSTRIPE)])
    pltpu.sync_copy(ones_hbm, ones_v)
    pltpu.sync_copy(dst_hbm.at[wid], idx_v)
    plsc.subcore_barrier()

    def body(b, carry):
        pltpu.sync_copy(ones_v, acc.at[idx_v.at[b]], add=True)
        return carry
    lax.fori_loop(0, D_NB, body, 0)
    plsc.subcore_barrier()
    pltpu.sync_copy(acc.at[pl.ds(row0, NSTRIPE)],
                    out_hbm.at[cid, pl.ds(row0, NSTRIPE)])


_kdeg = pl.kernel(
    _deg_body,
    out_type=jax.ShapeDtypeStruct((NC, NPAD, F), jnp.float32),
    mesh=_MESH,
    scratch_types=[
        pltpu.VMEM((D_NB, EB), jnp.int32),
        pltpu.VMEM((EB, F), jnp.float32),
        pltpu.VMEM_SHARED((NPAD, F), jnp.float32),
        pltpu.SemaphoreType.DMA,
    ],
)


def _edge_body(srcg_hbm, dste_hbm, hp_hbm, zeros_hbm, out_hbm,
               isrc_v, idst_v, gbuf_a, gbuf_b, acc, sem_a, sem_b):
    cid = lax.axis_index("c")
    sid = lax.axis_index("s")
    row0 = sid * NSTRIPE
    for j in range(NCHUNK // NC):
        ch = cid * (NCHUNK // NC) + j
        pltpu.sync_copy(zeros_hbm.at[pl.ds(row0, NSTRIPE)],
                        acc.at[pl.ds(row0, NSTRIPE)])
        plsc.subcore_barrier()
        for h in range(2):
            pltpu.sync_copy(srcg_hbm.at[ch, sid, pl.ds(h * E_NH, E_NH)],
                            isrc_v)
            pltpu.sync_copy(dste_hbm.at[sid, pl.ds(h * E_NH, E_NH)], idst_v)
            pltpu.async_copy(hp_hbm.at[isrc_v.at[0]], gbuf_a, sem_a)
            pltpu.async_copy(hp_hbm.at[isrc_v.at[1]], gbuf_b, sem_b)

            def body(g, carry):
                b = g * 2
                pltpu.make_async_copy(hp_hbm.at[isrc_v.at[b]],
                                      gbuf_a, sem_a).wait()
                pltpu.sync_copy(gbuf_a, acc.at[idst_v.at[b]], add=True)
                pltpu.async_copy(hp_hbm.at[isrc_v.at[lax.rem(b + 2, E_NH)]],
                                 gbuf_a, sem_a)
                pltpu.make_async_copy(hp_hbm.at[isrc_v.at[b + 1]],
                                      gbuf_b, sem_b).wait()
                pltpu.sync_copy(gbuf_b, acc.at[idst_v.at[b + 1]], add=True)
                pltpu.async_copy(hp_hbm.at[isrc_v.at[lax.rem(b + 3, E_NH)]],
                                 gbuf_b, sem_b)
                return carry
            lax.fori_loop(0, E_NH // 2, body, 0)
            pltpu.make_async_copy(hp_hbm.at[isrc_v.at[0]],
                                  gbuf_a, sem_a).wait()
            pltpu.make_async_copy(hp_hbm.at[isrc_v.at[1]],
                                  gbuf_b, sem_b).wait()
        plsc.subcore_barrier()
        col0 = pl.multiple_of(ch * F, F)
        pltpu.sync_copy(acc.at[pl.ds(row0, NSTRIPE)],
                        out_hbm.at[pl.ds(row0, NSTRIPE), pl.ds(col0, F)])


_kedge = pl.kernel(
    _edge_body,
    out_type=jax.ShapeDtypeStruct((NPAD, HID), jnp.float32),
    mesh=_MESH,
    scratch_types=[
        pltpu.VMEM((E_NH, EB), jnp.int32),
        pltpu.VMEM((E_NH, EB), jnp.int32),
        pltpu.VMEM((EB, F), jnp.float32),
        pltpu.VMEM((EB, F), jnp.float32),
        pltpu.VMEM_SHARED((NPAD, F), jnp.float32),
        pltpu.SemaphoreType.DMA,
        pltpu.SemaphoreType.DMA,
    ],
)


def _dec_compute(ga, gb, wv, res_v):
    for g in range(PB // 16):
        def kstep(k, accs):
            w = wv[pl.ds(k * 16, 16)]
            out = []
            for j in range(16):
                row = g * 16 + j
                av = ga[row, pl.ds(k * 16, 16)]
                bb = gb[row, pl.ds(k * 16, 16)]
                out.append(accs[j] + jnp.maximum(av + bb, 0.0) * w)
            return tuple(out)
        accs = lax.fori_loop(
            0, HID // 16, kstep,
            tuple(jnp.zeros((16,), jnp.float32) for _ in range(16)))
        for j in range(16):
            res_v[g * 16 + j] = accs[j]


def _dec_body(p0_hbm, p1_hbm, a_hbm, b_hbm, wc2_hbm, out_hbm,
              i0_v, i1_v, ga0, gb0, ga1, gb1, wv, res_v,
              sa0, sb0, sa1, sb1):
    cid = lax.axis_index("c")
    sid = lax.axis_index("s")
    wid = sid * NC + cid
    pltpu.sync_copy(p0_hbm.at[wid], i0_v)
    pltpu.sync_copy(p1_hbm.at[wid], i1_v)
    pltpu.sync_copy(wc2_hbm, wv)

    pltpu.async_copy(a_hbm.at[i0_v.at[0]], ga0, sa0)
    pltpu.async_copy(b_hbm.at[i1_v.at[0]], gb0, sb0)
    pltpu.async_copy(a_hbm.at[i0_v.at[1]], ga1, sa1)
    pltpu.async_copy(b_hbm.at[i1_v.at[1]], gb1, sb1)

    def batch(g, carry):
        b = g * 2
        pltpu.make_async_copy(a_hbm.at[i0_v.at[b]], ga0, sa0).wait()
        pltpu.make_async_copy(b_hbm.at[i1_v.at[b]], gb0, sb0).wait()
        _dec_compute(ga0, gb0, wv, res_v)
        pltpu.sync_copy(res_v, out_hbm.at[wid, pl.ds(b * PB, PB)])
        pltpu.async_copy(a_hbm.at[i0_v.at[lax.rem(b + 2, P_NB)]], ga0, sa0)
        pltpu.async_copy(b_hbm.at[i1_v.at[lax.rem(b + 2, P_NB)]], gb0, sb0)
        pltpu.make_async_copy(a_hbm.at[i0_v.at[b + 1]], ga1, sa1).wait()
        pltpu.make_async_copy(b_hbm.at[i1_v.at[b + 1]], gb1, sb1).wait()
        _dec_compute(ga1, gb1, wv, res_v)
        pltpu.sync_copy(res_v, out_hbm.at[wid, pl.ds((b + 1) * PB, PB)])
        pltpu.async_copy(a_hbm.at[i0_v.at[lax.rem(b + 3, P_NB)]], ga1, sa1)
        pltpu.async_copy(b_hbm.at[i1_v.at[lax.rem(b + 3, P_NB)]], gb1, sb1)
        return carry
    lax.fori_loop(0, P_NB // 2, batch, 0)
    pltpu.make_async_copy(a_hbm.at[i0_v.at[0]], ga0, sa0).wait()
    pltpu.make_async_copy(b_hbm.at[i1_v.at[0]], gb0, sb0).wait()
    pltpu.make_async_copy(a_hbm.at[i0_v.at[1]], ga1, sa1).wait()
    pltpu.make_async_copy(b_hbm.at[i1_v.at[1]], gb1, sb1).wait()


_kdec = pl.kernel(
    _dec_body,
    out_type=jax.ShapeDtypeStruct((NW, P_PER_W, 16), jnp.float32),
    mesh=_MESH,
    scratch_types=[
        pltpu.VMEM((P_NB, PB), jnp.int32),
        pltpu.VMEM((P_NB, PB), jnp.int32),
        pltpu.VMEM((PB, HID), jnp.float32),
        pltpu.VMEM((PB, HID), jnp.float32),
        pltpu.VMEM((PB, HID), jnp.float32),
        pltpu.VMEM((PB, HID), jnp.float32),
        pltpu.VMEM((HID,), jnp.float32),
        pltpu.VMEM((PB, 16), jnp.float32),
        pltpu.SemaphoreType.DMA,
        pltpu.SemaphoreType.DMA,
        pltpu.SemaphoreType.DMA,
        pltpu.SemaphoreType.DMA,
    ],
)


def _red_body(p_ref, bc2_ref, o_ref):
    o_ref[...] = jnp.sum(p_ref[...], axis=-1) + bc2_ref[0, 0]


_kred = pl.pallas_call(
    _red_body,
    grid=(NW // 8,),
    in_specs=[pl.BlockSpec((8, P_PER_W, 16), lambda i: (i, 0, 0)),
              pl.BlockSpec((1, 1), lambda i: (0, 0))],
    out_specs=pl.BlockSpec((8, P_PER_W), lambda i: (i, 0)),
    out_shape=jax.ShapeDtypeStruct((NW, P_PER_W), jnp.float32),
)


MBLK = 1000


def _dinv_block(deg3_ref, i):
    degb = deg3_ref[:, pl.ds(i * MBLK, MBLK), :]
    deg = jnp.sum(degb, axis=(0, 2)) * (1.0 / F) + 1.0
    return lax.rsqrt(jnp.maximum(deg, 1.0))


def _mm1_body(x_ref, w_ref, deg3_ref, o_ref):
    i = pl.program_id(0)
    dinv = _dinv_block(deg3_ref, i)
    h = jnp.dot(x_ref[...], w_ref[...], preferred_element_type=jnp.float32)
    o_ref[...] = h * dinv[:, None]


def _mm2_body(s_ref, hp_ref, w_ref, b_ref, deg3_ref, o_ref):
    i = pl.program_id(0)
    dinv = _dinv_block(deg3_ref, i)
    z = jax.nn.relu((s_ref[...] + hp_ref[...]) * dinv[:, None] + b_ref[...])
    h = jnp.dot(z, w_ref[...], preferred_element_type=jnp.float32)
    o_ref[...] = h * dinv[:, None]


def _mm3_body(s_ref, hp_ref, wc1_ref, b_ref, bc1_ref, deg3_ref,
              a_ref, b_out_ref):
    i = pl.program_id(0)
    dinv = _dinv_block(deg3_ref, i)
    z = (s_ref[...] + hp_ref[...]) * dinv[:, None] + b_ref[...]
    a_ref[...] = jnp.dot(z, wc1_ref[pl.ds(0, HID), :],
                         preferred_element_type=jnp.float32) + bc1_ref[...]
    b_out_ref[...] = jnp.dot(z, wc1_ref[pl.ds(HID, HID), :],
                             preferred_element_type=jnp.float32)


def _full_spec(shape):
    nd = len(shape)
    return pl.BlockSpec(shape, lambda i, _nd=nd: (0,) * _nd)


def _row_spec(cols):
    return pl.BlockSpec((MBLK, cols), lambda i: (i, 0))


_GRID = N // MBLK

_kmm1 = pl.pallas_call(
    _mm1_body,
    grid=(_GRID,),
    in_specs=[_row_spec(IN_CH), _full_spec((IN_CH, HID)),
              _full_spec((NC, NPAD, F))],
    out_specs=_row_spec(HID),
    out_shape=jax.ShapeDtypeStruct((N, HID), jnp.float32),
)

_kmm2 = pl.pallas_call(
    _mm2_body,
    grid=(_GRID,),
    in_specs=[_row_spec(HID), _row_spec(HID), _full_spec((HID, HID)),
              _full_spec((1, HID)), _full_spec((NC, NPAD, F))],
    out_specs=_row_spec(HID),
    out_shape=jax.ShapeDtypeStruct((N, HID), jnp.float32),
)

_kmm3 = pl.pallas_call(
    _mm3_body,
    grid=(_GRID,),
    in_specs=[_row_spec(HID), _row_spec(HID), _full_spec((2 * HID, HID)),
              _full_spec((1, HID)), _full_spec((1, HID)),
              _full_spec((NC, NPAD, F))],
    out_specs=[_row_spec(HID), _row_spec(HID)],
    out_shape=[jax.ShapeDtypeStruct((N, HID), jnp.float32),
               jax.ShapeDtypeStruct((N, HID), jnp.float32)],
)


def _prep_edges(edge_index):
    src = edge_index[0].astype(jnp.int32)
    dst = edge_index[1].astype(jnp.int32)
    epad = NS * E_PER_TILE
    src_p = jnp.pad(src, (0, epad - E), constant_values=0)
    dst_p = jnp.pad(dst, (0, epad - E), constant_values=N + 16)
    dst_deg = dst_p.reshape(NW, D_NB, EB)
    dst_edge = dst_p.reshape(NS, E_NB, EB)
    srcg = (src_p[None, :] * NCHUNK
            + jnp.arange(NCHUNK, dtype=jnp.int32)[:, None])
    srcg = srcg.reshape(NCHUNK, NS, E_NB, EB)
    return dst_deg, dst_edge, srcg


def kernel(x, edge_index, edge_pairs, W1, b1, W2, b2, Wc1, bc1, Wc2, bc2):
    dst_deg, dst_edge, srcg = _prep_edges(edge_index)
    p0 = edge_pairs[0].astype(jnp.int32).reshape(NW, P_REAL)
    p1 = edge_pairs[1].astype(jnp.int32).reshape(NW, P_REAL)
    p0 = jnp.pad(p0, ((0, 0), (0, P_PER_W - P_REAL))).reshape(NW, P_NB, PB)
    p1 = jnp.pad(p1, ((0, 0), (0, P_PER_W - P_REAL))).reshape(NW, P_NB, PB)

    zeros_big = jnp.zeros((NPAD, F), jnp.float32)
    ones128 = jnp.ones((EB, F), jnp.float32)
    b1r = b1.reshape(1, HID)
    b2r = b2.reshape(1, HID)
    bc1r = bc1.reshape(1, HID)
    wc2 = Wc2.reshape(HID)
    bc2r = bc2.reshape(1, 1)

    deg3 = _kdeg(dst_deg, ones128, zeros_big)
    h1p = _kmm1(x, W1, deg3)
    s1 = _kedge(srcg, dst_edge, h1p.reshape(N * NCHUNK, F), zeros_big)[:N]
    h2p = _kmm2(s1, h1p, W2, b1r, deg3)
    s2 = _kedge(srcg, dst_edge, h2p.reshape(N * NCHUNK, F), zeros_big)[:N]
    A, B = _kmm3(s2, h2p, Wc1, b2r, bc1r, deg3)
    outp = _kred(_kdec(p0, p1, A, B, wc2), bc2r)
    return outp[:, :P_REAL].reshape(-1)


# speedup vs baseline: 1.0947x; 1.0327x over previous
import functools

import jax
import jax.numpy as jnp
from jax import lax
from jax.experimental import pallas as pl
from jax.experimental.pallas import tpu as pltpu
from jax.experimental.pallas import tpu_sc as plsc

N = 10000
NPAD = 10240
E = 160000
P = 100000
IN_CH = 256
HID = 512
F = 128
NCHUNK = HID // F
NC, NS = 2, 16
NW = NC * NS

EB = 128
E_PER_TILE = 10240
E_NB = E_PER_TILE // EB
E_NH = E_NB // 2
E_PER_W = 5120
D_NB = E_PER_W // EB
NSTRIPE = NPAD // NS

PB = 32
P_PER_W = 3200
P_NB = P_PER_W // PB
P_REAL = P // NW

_MESH = plsc.VectorSubcoreMesh(core_axis_name="c", subcore_axis_name="s",
                               num_cores=NC, num_subcores=NS)


def _deg_body(dst_hbm, ones_hbm, zeros_hbm, out_hbm, idx_v, ones_v, acc, sem):
    cid = lax.axis_index("c")
    sid = lax.axis_index("s")
    wid = sid * NC + cid
    row0 = sid * NSTRIPE
    pltpu.sync_copy(zeros_hbm.at[pl.ds(row0, NSTRIPE)],
                    acc.at[pl.ds(row0, NSTRIPE)])
    pltpu.sync_copy(ones_hbm, ones_v)
    pltpu.sync_copy(dst_hbm.at[wid], idx_v)
    plsc.subcore_barrier()

    def body(b, carry):
        pltpu.sync_copy(ones_v, acc.at[idx_v.at[b]], add=True)
        return carry
    lax.fori_loop(0, D_NB, body, 0)
    plsc.subcore_barrier()
    pltpu.sync_copy(acc.at[pl.ds(row0, NSTRIPE)],
                    out_hbm.at[cid, pl.ds(row0, NSTRIPE)])


_kdeg = pl.kernel(
    _deg_body,
    out_type=jax.ShapeDtypeStruct((NC, NPAD, F), jnp.float32),
    mesh=_MESH,
    scratch_types=[
        pltpu.VMEM((D_NB, EB), jnp.int32),
        pltpu.VMEM((EB, F), jnp.float32),
        pltpu.VMEM_SHARED((NPAD, F), jnp.float32),
        pltpu.SemaphoreType.DMA,
    ],
)


def _edge_body(srcg_hbm, dste_hbm, hp_hbm, zeros_hbm, out_hbm,
               isrc_v, idst_v, gbuf_a, gbuf_b, acc, sem_a, sem_b):
    cid = lax.axis_index("c")
    sid = lax.axis_index("s")
    row0 = sid * NSTRIPE
    for j in range(NCHUNK // NC):
        ch = cid * (NCHUNK // NC) + j
        pltpu.sync_copy(zeros_hbm.at[pl.ds(row0, NSTRIPE)],
                        acc.at[pl.ds(row0, NSTRIPE)])
        plsc.subcore_barrier()
        for h in range(2):
            pltpu.sync_copy(srcg_hbm.at[ch, sid, pl.ds(h * E_NH, E_NH)],
                            isrc_v)
            pltpu.sync_copy(dste_hbm.at[sid, pl.ds(h * E_NH, E_NH)], idst_v)
            pltpu.async_copy(hp_hbm.at[isrc_v.at[0]], gbuf_a, sem_a)
            pltpu.async_copy(hp_hbm.at[isrc_v.at[1]], gbuf_b, sem_b)

            def body(g, carry):
                b = g * 2
                pltpu.make_async_copy(hp_hbm.at[isrc_v.at[b]],
                                      gbuf_a, sem_a).wait()
                pltpu.sync_copy(gbuf_a, acc.at[idst_v.at[b]], add=True)
                pltpu.async_copy(hp_hbm.at[isrc_v.at[lax.rem(b + 2, E_NH)]],
                                 gbuf_a, sem_a)
                pltpu.make_async_copy(hp_hbm.at[isrc_v.at[b + 1]],
                                      gbuf_b, sem_b).wait()
                pltpu.sync_copy(gbuf_b, acc.at[idst_v.at[b + 1]], add=True)
                pltpu.async_copy(hp_hbm.at[isrc_v.at[lax.rem(b + 3, E_NH)]],
                                 gbuf_b, sem_b)
                return carry
            lax.fori_loop(0, E_NH // 2, body, 0)
            pltpu.make_async_copy(hp_hbm.at[isrc_v.at[0]],
                                  gbuf_a, sem_a).wait()
            pltpu.make_async_copy(hp_hbm.at[isrc_v.at[1]],
                                  gbuf_b, sem_b).wait()
        plsc.subcore_barrier()
        col0 = pl.multiple_of(ch * F, F)
        pltpu.sync_copy(acc.at[pl.ds(row0, NSTRIPE)],
                        out_hbm.at[pl.ds(row0, NSTRIPE), pl.ds(col0, F)])


_kedge = pl.kernel(
    _edge_body,
    out_type=jax.ShapeDtypeStruct((NPAD, HID), jnp.float32),
    mesh=_MESH,
    scratch_types=[
        pltpu.VMEM((E_NH, EB), jnp.int32),
        pltpu.VMEM((E_NH, EB), jnp.int32),
        pltpu.VMEM((EB, F), jnp.float32),
        pltpu.VMEM((EB, F), jnp.float32),
        pltpu.VMEM_SHARED((NPAD, F), jnp.float32),
        pltpu.SemaphoreType.DMA,
        pltpu.SemaphoreType.DMA,
    ],
)


def _dec_compute(ga, gb, wv, res_v):
    for g in range(PB // 16):
        def kstep(k, accs):
            w = wv[pl.ds(k * 16, 16)]
            out = []
            for j in range(16):
                row = g * 16 + j
                av = ga[row, pl.ds(k * 16, 16)]
                bb = gb[row, pl.ds(k * 16, 16)]
                t = jnp.maximum(av + bb, 0.0)
                s = t * 65537.0
                t = s - (s - t)
                out.append(accs[j] + t * w)
            return tuple(out)
        accs = lax.fori_loop(
            0, HID // 16, kstep,
            tuple(jnp.zeros((16,), jnp.float32) for _ in range(16)))
        for j in range(16):
            res_v[g * 16 + j] = accs[j]


def _dec_body(p0_hbm, p1_hbm, a_hbm, b_hbm, wc2_hbm, out_hbm,
              i0_v, i1_v, ga0, gb0, ga1, gb1, wv, res_v,
              sa0, sb0, sa1, sb1):
    cid = lax.axis_index("c")
    sid = lax.axis_index("s")
    wid = sid * NC + cid
    pltpu.sync_copy(p0_hbm.at[wid], i0_v)
    pltpu.sync_copy(p1_hbm.at[wid], i1_v)
    pltpu.sync_copy(wc2_hbm, wv)

    pltpu.async_copy(a_hbm.at[i0_v.at[0]], ga0, sa0)
    pltpu.async_copy(b_hbm.at[i1_v.at[0]], gb0, sb0)
    pltpu.async_copy(a_hbm.at[i0_v.at[1]], ga1, sa1)
    pltpu.async_copy(b_hbm.at[i1_v.at[1]], gb1, sb1)

    def batch(g, carry):
        b = g * 2
        pltpu.make_async_copy(a_hbm.at[i0_v.at[b]], ga0, sa0).wait()
        pltpu.make_async_copy(b_hbm.at[i1_v.at[b]], gb0, sb0).wait()
        _dec_compute(ga0, gb0, wv, res_v)
        pltpu.sync_copy(res_v, out_hbm.at[wid, pl.ds(b * PB, PB)])
        pltpu.async_copy(a_hbm.at[i0_v.at[lax.rem(b + 2, P_NB)]], ga0, sa0)
        pltpu.async_copy(b_hbm.at[i1_v.at[lax.rem(b + 2, P_NB)]], gb0, sb0)
        pltpu.make_async_copy(a_hbm.at[i0_v.at[b + 1]], ga1, sa1).wait()
        pltpu.make_async_copy(b_hbm.at[i1_v.at[b + 1]], gb1, sb1).wait()
        _dec_compute(ga1, gb1, wv, res_v)
        pltpu.sync_copy(res_v, out_hbm.at[wid, pl.ds((b + 1) * PB, PB)])
        pltpu.async_copy(a_hbm.at[i0_v.at[lax.rem(b + 3, P_NB)]], ga1, sa1)
        pltpu.async_copy(b_hbm.at[i1_v.at[lax.rem(b + 3, P_NB)]], gb1, sb1)
        return carry
    lax.fori_loop(0, P_NB // 2, batch, 0)
    pltpu.make_async_copy(a_hbm.at[i0_v.at[0]], ga0, sa0).wait()
    pltpu.make_async_copy(b_hbm.at[i1_v.at[0]], gb0, sb0).wait()
    pltpu.make_async_copy(a_hbm.at[i0_v.at[1]], ga1, sa1).wait()
    pltpu.make_async_copy(b_hbm.at[i1_v.at[1]], gb1, sb1).wait()


_kdec = pl.kernel(
    _dec_body,
    out_type=jax.ShapeDtypeStruct((NW, P_PER_W, 16), jnp.float32),
    mesh=_MESH,
    scratch_types=[
        pltpu.VMEM((P_NB, PB), jnp.int32),
        pltpu.VMEM((P_NB, PB), jnp.int32),
        pltpu.VMEM((PB, HID), jnp.float32),
        pltpu.VMEM((PB, HID), jnp.float32),
        pltpu.VMEM((PB, HID), jnp.float32),
        pltpu.VMEM((PB, HID), jnp.float32),
        pltpu.VMEM((HID,), jnp.float32),
        pltpu.VMEM((PB, 16), jnp.float32),
        pltpu.SemaphoreType.DMA,
        pltpu.SemaphoreType.DMA,
        pltpu.SemaphoreType.DMA,
        pltpu.SemaphoreType.DMA,
    ],
)


def _red_body(p_ref, bc2_ref, o_ref):
    o_ref[...] = jnp.sum(p_ref[...], axis=-1) + bc2_ref[0, 0]


_kred = pl.pallas_call(
    _red_body,
    grid=(NW // 8,),
    in_specs=[pl.BlockSpec((8, P_PER_W, 16), lambda i: (i, 0, 0)),
              pl.BlockSpec((1, 1), lambda i: (0, 0))],
    out_specs=pl.BlockSpec((8, P_PER_W), lambda i: (i, 0)),
    out_shape=jax.ShapeDtypeStruct((NW, P_PER_W), jnp.float32),
)


MBLK = 1000


def _dinv_block(deg3_ref, i):
    degb = deg3_ref[:, pl.ds(i * MBLK, MBLK), :]
    deg = jnp.sum(degb, axis=(0, 2)) * (1.0 / F) + 1.0
    return lax.rsqrt(jnp.maximum(deg, 1.0))


def _mm1_body(x_ref, w_ref, deg3_ref, o_ref):
    i = pl.program_id(0)
    dinv = _dinv_block(deg3_ref, i)
    h = jnp.dot(x_ref[...], w_ref[...], preferred_element_type=jnp.float32)
    o_ref[...] = h * dinv[:, None]


def _mm2_body(s_ref, hp_ref, w_ref, b_ref, deg3_ref, o_ref):
    i = pl.program_id(0)
    dinv = _dinv_block(deg3_ref, i)
    z = jax.nn.relu((s_ref[...] + hp_ref[...]) * dinv[:, None] + b_ref[...])
    h = jnp.dot(z, w_ref[...], preferred_element_type=jnp.float32)
    o_ref[...] = h * dinv[:, None]


def _mm3_body(s_ref, hp_ref, wc1_ref, b_ref, bc1_ref, deg3_ref,
              a_ref, b_out_ref):
    i = pl.program_id(0)
    dinv = _dinv_block(deg3_ref, i)
    z = (s_ref[...] + hp_ref[...]) * dinv[:, None] + b_ref[...]
    zb = z.astype(jnp.bfloat16)
    w1b = wc1_ref[pl.ds(0, HID), :].astype(jnp.bfloat16)
    w2b = wc1_ref[pl.ds(HID, HID), :].astype(jnp.bfloat16)
    a_ref[...] = jnp.dot(zb, w1b,
                         preferred_element_type=jnp.float32) + bc1_ref[...]
    b_out_ref[...] = jnp.dot(zb, w2b, preferred_element_type=jnp.float32)


def _full_spec(shape):
    nd = len(shape)
    return pl.BlockSpec(shape, lambda i, _nd=nd: (0,) * _nd)


def _row_spec(cols):
    return pl.BlockSpec((MBLK, cols), lambda i: (i, 0))


_GRID = N // MBLK

_kmm1 = pl.pallas_call(
    _mm1_body,
    grid=(_GRID,),
    in_specs=[_row_spec(IN_CH), _full_spec((IN_CH, HID)),
              _full_spec((NC, NPAD, F))],
    out_specs=_row_spec(HID),
    out_shape=jax.ShapeDtypeStruct((N, HID), jnp.float32),
)

_kmm2 = pl.pallas_call(
    _mm2_body,
    grid=(_GRID,),
    in_specs=[_row_spec(HID), _row_spec(HID), _full_spec((HID, HID)),
              _full_spec((1, HID)), _full_spec((NC, NPAD, F))],
    out_specs=_row_spec(HID),
    out_shape=jax.ShapeDtypeStruct((N, HID), jnp.float32),
)

_kmm3 = pl.pallas_call(
    _mm3_body,
    grid=(_GRID,),
    in_specs=[_row_spec(HID), _row_spec(HID), _full_spec((2 * HID, HID)),
              _full_spec((1, HID)), _full_spec((1, HID)),
              _full_spec((NC, NPAD, F))],
    out_specs=[_row_spec(HID), _row_spec(HID)],
    out_shape=[jax.ShapeDtypeStruct((N, HID), jnp.float32),
               jax.ShapeDtypeStruct((N, HID), jnp.float32)],
)


def _prep_edges(edge_index):
    src = edge_index[0].astype(jnp.int32)
    dst = edge_index[1].astype(jnp.int32)
    epad = NS * E_PER_TILE
    src_p = jnp.pad(src, (0, epad - E), constant_values=0)
    dst_p = jnp.pad(dst, (0, epad - E), constant_values=N + 16)
    dst_deg = dst_p.reshape(NW, D_NB, EB)
    dst_edge = dst_p.reshape(NS, E_NB, EB)
    srcg = (src_p[None, :] * NCHUNK
            + jnp.arange(NCHUNK, dtype=jnp.int32)[:, None])
    srcg = srcg.reshape(NCHUNK, NS, E_NB, EB)
    return dst_deg, dst_edge, srcg


def kernel(x, edge_index, edge_pairs, W1, b1, W2, b2, Wc1, bc1, Wc2, bc2):
    dst_deg, dst_edge, srcg = _prep_edges(edge_index)
    p0 = edge_pairs[0].astype(jnp.int32).reshape(NW, P_REAL)
    p1 = edge_pairs[1].astype(jnp.int32).reshape(NW, P_REAL)
    p0 = jnp.pad(p0, ((0, 0), (0, P_PER_W - P_REAL))).reshape(NW, P_NB, PB)
    p1 = jnp.pad(p1, ((0, 0), (0, P_PER_W - P_REAL))).reshape(NW, P_NB, PB)

    zeros_big = jnp.zeros((NPAD, F), jnp.float32)
    ones128 = jnp.ones((EB, F), jnp.float32)
    b1r = b1.reshape(1, HID)
    b2r = b2.reshape(1, HID)
    bc1r = bc1.reshape(1, HID)
    wc2 = lax.reduce_precision(Wc2.reshape(HID), 8, 7)
    bc2r = bc2.reshape(1, 1)

    deg3 = _kdeg(dst_deg, ones128, zeros_big)
    h1p = _kmm1(x, W1, deg3)
    s1 = _kedge(srcg, dst_edge, h1p.reshape(N * NCHUNK, F), zeros_big)[:N]
    h2p = _kmm2(s1, h1p, W2, b1r, deg3)
    s2 = _kedge(srcg, dst_edge, h2p.reshape(N * NCHUNK, F), zeros_big)[:N]
    A, B = _kmm3(s2, h2p, Wc1, b2r, bc1r, deg3)
    outp = _kred(_kdec(p0, p1, A, B, wc2), bc2r)
    return outp[:, :P_REAL].reshape(-1)
